# baseline (device time: 1634275 ns/iter reference)
import jax
import jax.numpy as jnp
from jax import lax
from jax.experimental import pallas as pl
from jax.experimental.pallas import tpu as pltpu

N_DEV = 32
LOG2_N = 5

_sem_signal = getattr(pl, "semaphore_signal", None) or pltpu.semaphore_signal
_sem_wait = getattr(pl, "semaphore_wait", None) or pltpu.semaphore_wait


def kernel(x, w_mat):
    m_glob, k_loc = x.shape
    _, n_col = w_mat.shape
    m_blk = m_glob // N_DEV

    def body(x_ref, w_ref, out_ref, acc_ref, send_sems, recv_sems,
             credit_sems, ax_send, ax_recv, ax_send_sems, ax_recv_sems):
        d = lax.axis_index("i")
        left = lax.rem(d + N_DEV - 1, N_DEV)
        right = lax.rem(d + 1, N_DEV)

        barrier_sem = pltpu.get_barrier_semaphore()
        for nbr in (left, right):
            _sem_signal(barrier_sem, inc=1, device_id=(nbr,),
                        device_id_type=pl.DeviceIdType.MESH)
        _sem_wait(barrier_sem, 2)

        def chunk(j):
            xc = x_ref[pl.ds(j * m_blk, m_blk), :]
            return lax.dot_general(
                xc, w_ref[:, :], (((1,), (0,)), ((), ())),
                precision=lax.Precision.HIGHEST,
                preferred_element_type=jnp.float32,
            )

        acc_ref[0, :, :] = chunk(lax.rem(d + N_DEV - 1, N_DEV))

        y = None
        for s in range(1, N_DEV):
            src = (s - 1) % 2
            dst = s % 2
            if s >= 2:
                _sem_wait(credit_sems.at[dst], 1)
            rdma = pltpu.make_async_remote_copy(
                src_ref=acc_ref.at[src],
                dst_ref=acc_ref.at[dst],
                send_sem=send_sems.at[src],
                recv_sem=recv_sems.at[dst],
                device_id=(right,),
                device_id_type=pl.DeviceIdType.MESH,
            )
            rdma.start()
            j = lax.rem(d - s - 1 + 2 * N_DEV, N_DEV)
            c = chunk(j)
            rdma.wait_recv()
            rdma.wait_send()
            if s <= N_DEV - 2:
                _sem_signal(credit_sems.at[src], inc=1, device_id=(left,),
                            device_id_type=pl.DeviceIdType.MESH)
            if s < N_DEV - 1:
                acc_ref[dst, :, :] = acc_ref[dst, :, :] + c
            else:
                y = acc_ref[dst, :, :] + c
        out_ref[:, :] = y

        amax = jnp.max(jnp.abs(y))
        for k in range(LOG2_N):
            partner = jnp.bitwise_xor(d, 1 << k)
            ax_send[k, :, :] = jnp.full((8, 128), amax, jnp.float32)
            ax = pltpu.make_async_remote_copy(
                src_ref=ax_send.at[k],
                dst_ref=ax_recv.at[k],
                send_sem=ax_send_sems.at[k],
                recv_sem=ax_recv_sems.at[k],
                device_id=(partner,),
                device_id_type=pl.DeviceIdType.MESH,
            )
            ax.start()
            ax.wait()
            amax = jnp.maximum(amax, jnp.max(ax_recv[k, :, :]))

        scale = amax / 127.0
        q = jnp.clip(jnp.round(out_ref[:, :] / scale), -127.0, 127.0)
        out_ref[:, :] = q * scale

    return pl.pallas_call(
        body,
        out_shape=jax.ShapeDtypeStruct((m_blk, n_col), jnp.float32),
        in_specs=[
            pl.BlockSpec(memory_space=pltpu.VMEM),
            pl.BlockSpec(memory_space=pltpu.VMEM),
        ],
        out_specs=pl.BlockSpec(memory_space=pltpu.VMEM),
        scratch_shapes=[
            pltpu.VMEM((2, m_blk, n_col), jnp.float32),
            pltpu.SemaphoreType.DMA((2,)),
            pltpu.SemaphoreType.DMA((2,)),
            pltpu.SemaphoreType.REGULAR((2,)),
            pltpu.VMEM((LOG2_N, 8, 128), jnp.float32),
            pltpu.VMEM((LOG2_N, 8, 128), jnp.float32),
            pltpu.SemaphoreType.DMA((LOG2_N,)),
            pltpu.SemaphoreType.DMA((LOG2_N,)),
        ],
        compiler_params=pltpu.CompilerParams(collective_id=0),
    )(x, w_mat)


# device time: 1518662 ns/iter; 1.0761x vs baseline; 1.0761x over previous
import jax
import jax.numpy as jnp
from jax import lax
from jax.experimental import pallas as pl
from jax.experimental.pallas import tpu as pltpu

N_DEV = 32
LOG2_N = 5

_sem_signal = getattr(pl, "semaphore_signal", None) or pltpu.semaphore_signal
_sem_wait = getattr(pl, "semaphore_wait", None) or pltpu.semaphore_wait


def kernel(x, w_mat):
    m_glob, k_loc = x.shape
    _, n_col = w_mat.shape
    m_blk = m_glob // N_DEV
    n_half = n_col // 2

    def body(x_ref, w_ref, out_ref, acc_r, acc_l, send_sems, recv_sems,
             credit_sems, ax_send, ax_recv, ax_send_sems, ax_recv_sems):
        d = lax.axis_index("i")
        left = lax.rem(d + N_DEV - 1, N_DEV)
        right = lax.rem(d + 1, N_DEV)

        barrier_sem = pltpu.get_barrier_semaphore()
        for nbr in (left, right):
            _sem_signal(barrier_sem, inc=1, device_id=(nbr,),
                        device_id_type=pl.DeviceIdType.MESH)
        _sem_wait(barrier_sem, 2)

        def chunk(j, col0):
            xc = x_ref[pl.ds(j * m_blk, m_blk), :]
            return lax.dot_general(
                xc, w_ref[:, col0:col0 + n_half], (((1,), (0,)), ((), ())),
                precision=lax.Precision.HIGHEST,
                preferred_element_type=jnp.float32,
            )

        rings = (
            dict(acc=acc_r, col0=0, to=right, writer=left, jsign=-1),
            dict(acc=acc_l, col0=n_half, to=left, writer=right, jsign=+1),
        )

        for rg in rings:
            j0 = lax.rem(d + rg["jsign"] + N_DEV, N_DEV)
            rg["acc"][0, :, :] = chunk(j0, rg["col0"])

        for s in range(1, N_DEV):
            src = (s - 1) % 2
            dst = s % 2
            rdmas = []
            for r, rg in enumerate(rings):
                if s >= 2:
                    _sem_wait(credit_sems.at[r, dst], 1)
                rdma = pltpu.make_async_remote_copy(
                    src_ref=rg["acc"].at[src],
                    dst_ref=rg["acc"].at[dst],
                    send_sem=send_sems.at[r, src],
                    recv_sem=recv_sems.at[r, dst],
                    device_id=(rg["to"],),
                    device_id_type=pl.DeviceIdType.MESH,
                )
                rdma.start()
                rdmas.append(rdma)
            cs = [
                chunk(lax.rem(d + rg["jsign"] * (s + 1) + 2 * N_DEV, N_DEV),
                      rg["col0"])
                for rg in rings
            ]
            for r, (rg, rdma, c) in enumerate(zip(rings, rdmas, cs)):
                rdma.wait_recv()
                rdma.wait_send()
                if s <= N_DEV - 2:
                    _sem_signal(credit_sems.at[r, src], inc=1,
                                device_id=(rg["writer"],),
                                device_id_type=pl.DeviceIdType.MESH)
                if s < N_DEV - 1:
                    rg["acc"][dst, :, :] = rg["acc"][dst, :, :] + c
                else:
                    out_ref[:, rg["col0"]:rg["col0"] + n_half] = (
                        rg["acc"][dst, :, :] + c
                    )

        amax = jnp.max(jnp.abs(out_ref[:, :]))
        for k in range(LOG2_N):
            partner = jnp.bitwise_xor(d, 1 << k)
            ax_send[k, :, :] = jnp.full((8, 128), amax, jnp.float32)
            ax = pltpu.make_async_remote_copy(
                src_ref=ax_send.at[k],
                dst_ref=ax_recv.at[k],
                send_sem=ax_send_sems.at[k],
                recv_sem=ax_recv_sems.at[k],
                device_id=(partner,),
                device_id_type=pl.DeviceIdType.MESH,
            )
            ax.start()
            ax.wait()
            amax = jnp.maximum(amax, jnp.max(ax_recv[k, :, :]))

        scale = amax / 127.0
        q = jnp.clip(jnp.round(out_ref[:, :] / scale), -127.0, 127.0)
        out_ref[:, :] = q * scale

    return pl.pallas_call(
        body,
        out_shape=jax.ShapeDtypeStruct((m_blk, n_col), jnp.float32),
        in_specs=[
            pl.BlockSpec(memory_space=pltpu.VMEM),
            pl.BlockSpec(memory_space=pltpu.VMEM),
        ],
        out_specs=pl.BlockSpec(memory_space=pltpu.VMEM),
        scratch_shapes=[
            pltpu.VMEM((2, m_blk, n_half), jnp.float32),
            pltpu.VMEM((2, m_blk, n_half), jnp.float32),
            pltpu.SemaphoreType.DMA((2, 2)),
            pltpu.SemaphoreType.DMA((2, 2)),
            pltpu.SemaphoreType.REGULAR((2, 2)),
            pltpu.VMEM((LOG2_N, 8, 128), jnp.float32),
            pltpu.VMEM((LOG2_N, 8, 128), jnp.float32),
            pltpu.SemaphoreType.DMA((LOG2_N,)),
            pltpu.SemaphoreType.DMA((LOG2_N,)),
        ],
        compiler_params=pltpu.CompilerParams(collective_id=0),
    )(x, w_mat)


# device time: 798383 ns/iter; 2.0470x vs baseline; 1.9022x over previous
import jax
import jax.numpy as jnp
from jax import lax
from jax.experimental import pallas as pl
from jax.experimental.pallas import tpu as pltpu

N_DEV = 32
LOG2_N = 5

_sem_signal = getattr(pl, "semaphore_signal", None) or pltpu.semaphore_signal
_sem_wait = getattr(pl, "semaphore_wait", None) or pltpu.semaphore_wait


def _coords_of_pos(p):
    z = p // 8
    idx = p % 8
    y = idx // 2
    xb = idx % 2
    x = jnp.where(y % 2 == 0, xb, 1 - xb)
    return x, y, z


def _rank_of_pos(p):
    x, y, z = _coords_of_pos(p)
    pr = 4 * z + jnp.where(z % 2 == 0, y, 3 - y)
    return jnp.where(x == 0, pr, 31 - pr)


def _pos_of_rank(rr):
    rr = jnp.remainder(rr, N_DEV)
    x = jnp.where(rr < 16, 0, 1)
    pr = jnp.where(rr < 16, rr, 31 - rr)
    z = pr // 4
    yy = pr % 4
    y = jnp.where(z % 2 == 0, yy, 3 - yy)
    idx = 2 * y + jnp.where(y % 2 == 0, x, 1 - x)
    return 8 * z + idx


def kernel(x, w_mat):
    m_glob, k_loc = x.shape
    _, n_col = w_mat.shape
    m_blk = m_glob // N_DEV
    n_half = n_col // 2

    def body(x_ref, w_ref, out_ref, acc_r, acc_l, send_sems, recv_sems,
             credit_sems, ax_send, ax_recv, ax_send_sems, ax_recv_sems):
        d = lax.axis_index("i")
        rk = _rank_of_pos(d)
        fwd = _pos_of_rank(rk + 1)
        bwd = _pos_of_rank(rk + N_DEV - 1)

        barrier_sem = pltpu.get_barrier_semaphore()
        for nbr in (bwd, fwd):
            _sem_signal(barrier_sem, inc=1, device_id=(nbr,),
                        device_id_type=pl.DeviceIdType.MESH)
        _sem_wait(barrier_sem, 2)

        def chunk(j, col0):
            xc = x_ref[pl.ds(j * m_blk, m_blk), :]
            return lax.dot_general(
                xc, w_ref[:, col0:col0 + n_half], (((1,), (0,)), ((), ())),
                precision=lax.Precision.HIGHEST,
                preferred_element_type=jnp.float32,
            )

        rings = (
            dict(acc=acc_r, col0=0, to=fwd, writer=bwd, jsign=-1),
            dict(acc=acc_l, col0=n_half, to=bwd, writer=fwd, jsign=+1),
        )

        for rg in rings:
            rg["acc"][0, :, :] = chunk(
                _pos_of_rank(rk + rg["jsign"] + N_DEV), rg["col0"])

        for s in range(1, N_DEV):
            src = (s - 1) % 2
            dst = s % 2
            rdmas = []
            for r, rg in enumerate(rings):
                if s >= 2:
                    _sem_wait(credit_sems.at[r, dst], 1)
                rdma = pltpu.make_async_remote_copy(
                    src_ref=rg["acc"].at[src],
                    dst_ref=rg["acc"].at[dst],
                    send_sem=send_sems.at[r, src],
                    recv_sem=recv_sems.at[r, dst],
                    device_id=(rg["to"],),
                    device_id_type=pl.DeviceIdType.MESH,
                )
                rdma.start()
                rdmas.append(rdma)
            cs = [
                chunk(_pos_of_rank(rk + rg["jsign"] * (s + 1) + 2 * N_DEV),
                      rg["col0"])
                for rg in rings
            ]
            for r, (rg, rdma, c) in enumerate(zip(rings, rdmas, cs)):
                rdma.wait_recv()
                rdma.wait_send()
                if s <= N_DEV - 2:
                    _sem_signal(credit_sems.at[r, src], inc=1,
                                device_id=(rg["writer"],),
                                device_id_type=pl.DeviceIdType.MESH)
                if s < N_DEV - 1:
                    rg["acc"][dst, :, :] = rg["acc"][dst, :, :] + c
                else:
                    out_ref[:, rg["col0"]:rg["col0"] + n_half] = (
                        rg["acc"][dst, :, :] + c
                    )

        amax = jnp.max(jnp.abs(out_ref[:, :]))
        for k in range(LOG2_N):
            partner = jnp.bitwise_xor(d, 1 << k)
            ax_send[k, :, :] = jnp.full((8, 128), amax, jnp.float32)
            ax = pltpu.make_async_remote_copy(
                src_ref=ax_send.at[k],
                dst_ref=ax_recv.at[k],
                send_sem=ax_send_sems.at[k],
                recv_sem=ax_recv_sems.at[k],
                device_id=(partner,),
                device_id_type=pl.DeviceIdType.MESH,
            )
            ax.start()
            ax.wait()
            amax = jnp.maximum(amax, jnp.max(ax_recv[k, :, :]))

        scale = amax / 127.0
        q = jnp.clip(jnp.round(out_ref[:, :] / scale), -127.0, 127.0)
        out_ref[:, :] = q * scale

    return pl.pallas_call(
        body,
        out_shape=jax.ShapeDtypeStruct((m_blk, n_col), jnp.float32),
        in_specs=[
            pl.BlockSpec(memory_space=pltpu.VMEM),
            pl.BlockSpec(memory_space=pltpu.VMEM),
        ],
        out_specs=pl.BlockSpec(memory_space=pltpu.VMEM),
        scratch_shapes=[
            pltpu.VMEM((2, m_blk, n_half), jnp.float32),
            pltpu.VMEM((2, m_blk, n_half), jnp.float32),
            pltpu.SemaphoreType.DMA((2, 2)),
            pltpu.SemaphoreType.DMA((2, 2)),
            pltpu.SemaphoreType.REGULAR((2, 2)),
            pltpu.VMEM((LOG2_N, 8, 128), jnp.float32),
            pltpu.VMEM((LOG2_N, 8, 128), jnp.float32),
            pltpu.SemaphoreType.DMA((LOG2_N,)),
            pltpu.SemaphoreType.DMA((LOG2_N,)),
        ],
        compiler_params=pltpu.CompilerParams(collective_id=0),
    )(x, w_mat)


# device time: 726408 ns/iter; 2.2498x vs baseline; 1.0991x over previous
import jax
import jax.numpy as jnp
from jax import lax
from jax.experimental import pallas as pl
from jax.experimental.pallas import tpu as pltpu

N_DEV = 32
LOG2_N = 5

_sem_signal = getattr(pl, "semaphore_signal", None) or pltpu.semaphore_signal
_sem_wait = getattr(pl, "semaphore_wait", None) or pltpu.semaphore_wait


def _coords_of_pos(p):
    z = p // 8
    idx = p % 8
    y = idx // 2
    xb = idx % 2
    x = jnp.where(y % 2 == 0, xb, 1 - xb)
    return x, y, z


def _rank_of_pos(p):
    x, y, z = _coords_of_pos(p)
    pr = 4 * z + jnp.where(z % 2 == 0, y, 3 - y)
    return jnp.where(x == 0, pr, 31 - pr)


def _pos_of_rank(rr):
    rr = jnp.remainder(rr, N_DEV)
    x = jnp.where(rr < 16, 0, 1)
    pr = jnp.where(rr < 16, rr, 31 - rr)
    z = pr // 4
    yy = pr % 4
    y = jnp.where(z % 2 == 0, yy, 3 - yy)
    idx = 2 * y + jnp.where(y % 2 == 0, x, 1 - x)
    return 8 * z + idx


def kernel(x, w_mat):
    m_glob, k_loc = x.shape
    _, n_col = w_mat.shape
    m_blk = m_glob // N_DEV
    n_half = n_col // 2

    n_q = n_half // 2

    def body(x_ref, w_ref, out_ref, acc_ref, send_sems, recv_sems,
             credit_sems, ax_send, ax_recv, ax_send_sems, ax_recv_sems):
        d = lax.axis_index("i")
        rk = _rank_of_pos(d)
        fwd = _pos_of_rank(rk + 1)
        bwd = _pos_of_rank(rk + N_DEV - 1)

        barrier_sem = pltpu.get_barrier_semaphore()
        for nbr in (bwd, fwd):
            _sem_signal(barrier_sem, inc=1, device_id=(nbr,),
                        device_id_type=pl.DeviceIdType.MESH)
        _sem_wait(barrier_sem, 2)

        def chunk_half(j, col0):
            xc = x_ref[pl.ds(j * m_blk, m_blk), :]
            return lax.dot_general(
                xc, w_ref[:, col0:col0 + n_half], (((1,), (0,)), ((), ())),
                precision=lax.Precision.HIGHEST,
                preferred_element_type=jnp.float32,
            )

        rings = (
            dict(col0=0, to=fwd, writer=bwd, jsign=-1),
            dict(col0=n_half, to=bwd, writer=fwd, jsign=+1),
        )
        stripes = []
        for q in range(2):
            for r, rg in enumerate(rings):
                stripes.append(dict(
                    sid=2 * q + r, r=r, q=q, rg=rg,
                    col0=rg["col0"] + q * n_q,
                ))

        def mk_rdma(st, s):
            return pltpu.make_async_remote_copy(
                src_ref=acc_ref.at[st["sid"], (s - 1) % 2],
                dst_ref=acc_ref.at[st["sid"], s % 2],
                send_sem=send_sems.at[st["sid"], (s - 1) % 2],
                recv_sem=recv_sems.at[st["sid"], s % 2],
                device_id=(st["rg"]["to"],),
                device_id_type=pl.DeviceIdType.MESH,
            )

        cseed = [
            chunk_half(_pos_of_rank(rk + rg["jsign"] + N_DEV), rg["col0"])
            for rg in rings
        ]
        for st in stripes:
            acc_ref[st["sid"], 0, :, :] = (
                cseed[st["r"]][:, st["q"] * n_q:(st["q"] + 1) * n_q])

        rdmas = {}
        for st in stripes:
            rdmas[st["sid"]] = mk_rdma(st, 1)
            rdmas[st["sid"]].start()

        for s in range(1, N_DEV):
            src = (s - 1) % 2
            dst = s % 2
            cs = [
                chunk_half(
                    _pos_of_rank(rk + rg["jsign"] * (s + 1) + 2 * N_DEV),
                    rg["col0"])
                for rg in rings
            ]
            for st in stripes:
                rdma = rdmas[st["sid"]]
                rdma.wait_recv()
                rdma.wait_send()
                if s <= N_DEV - 2:
                    _sem_signal(credit_sems.at[st["sid"], src], inc=1,
                                device_id=(st["rg"]["writer"],),
                                device_id_type=pl.DeviceIdType.MESH)
                c = cs[st["r"]][:, st["q"] * n_q:(st["q"] + 1) * n_q]
                if s < N_DEV - 1:
                    acc_ref[st["sid"], dst, :, :] = (
                        acc_ref[st["sid"], dst, :, :] + c)
                    _sem_wait(credit_sems.at[st["sid"], src], 1)
                    nxt = mk_rdma(st, s + 1)
                    nxt.start()
                    rdmas[st["sid"]] = nxt
                else:
                    out_ref[:, st["col0"]:st["col0"] + n_q] = (
                        acc_ref[st["sid"], dst, :, :] + c)

        amax = jnp.max(jnp.abs(out_ref[:, :]))
        for k in range(LOG2_N):
            partner = jnp.bitwise_xor(d, 1 << k)
            ax_send[k, :, :] = jnp.full((8, 128), amax, jnp.float32)
            ax = pltpu.make_async_remote_copy(
                src_ref=ax_send.at[k],
                dst_ref=ax_recv.at[k],
                send_sem=ax_send_sems.at[k],
                recv_sem=ax_recv_sems.at[k],
                device_id=(partner,),
                device_id_type=pl.DeviceIdType.MESH,
            )
            ax.start()
            ax.wait()
            amax = jnp.maximum(amax, jnp.max(ax_recv[k, :, :]))

        scale = amax / 127.0
        q = jnp.clip(jnp.round(out_ref[:, :] / scale), -127.0, 127.0)
        out_ref[:, :] = q * scale

    return pl.pallas_call(
        body,
        out_shape=jax.ShapeDtypeStruct((m_blk, n_col), jnp.float32),
        in_specs=[
            pl.BlockSpec(memory_space=pltpu.VMEM),
            pl.BlockSpec(memory_space=pltpu.VMEM),
        ],
        out_specs=pl.BlockSpec(memory_space=pltpu.VMEM),
        scratch_shapes=[
            pltpu.VMEM((4, 2, m_blk, n_half // 2), jnp.float32),
            pltpu.SemaphoreType.DMA((4, 2)),
            pltpu.SemaphoreType.DMA((4, 2)),
            pltpu.SemaphoreType.REGULAR((4, 2)),
            pltpu.VMEM((LOG2_N, 8, 128), jnp.float32),
            pltpu.VMEM((LOG2_N, 8, 128), jnp.float32),
            pltpu.SemaphoreType.DMA((LOG2_N,)),
            pltpu.SemaphoreType.DMA((LOG2_N,)),
        ],
        compiler_params=pltpu.CompilerParams(collective_id=0),
    )(x, w_mat)


# device time: 721078 ns/iter; 2.2664x vs baseline; 1.0074x over previous
import jax
import jax.numpy as jnp
from jax import lax
from jax.experimental import pallas as pl
from jax.experimental.pallas import tpu as pltpu

N_DEV = 32
LOG2_N = 5

_sem_signal = getattr(pl, "semaphore_signal", None) or pltpu.semaphore_signal
_sem_wait = getattr(pl, "semaphore_wait", None) or pltpu.semaphore_wait


def _coords_of_pos(p):
    z = p // 8
    idx = p % 8
    y = idx // 2
    xb = idx % 2
    x = jnp.where(y % 2 == 0, xb, 1 - xb)
    return x, y, z


def _rank_of_pos(p):
    x, y, z = _coords_of_pos(p)
    pr = 4 * z + jnp.where(z % 2 == 0, y, 3 - y)
    return jnp.where(x == 0, pr, 31 - pr)


def _pos_of_rank(rr):
    rr = jnp.remainder(rr, N_DEV)
    x = jnp.where(rr < 16, 0, 1)
    pr = jnp.where(rr < 16, rr, 31 - rr)
    z = pr // 4
    yy = pr % 4
    y = jnp.where(z % 2 == 0, yy, 3 - yy)
    idx = 2 * y + jnp.where(y % 2 == 0, x, 1 - x)
    return 8 * z + idx


def kernel(x, w_mat):
    m_glob, k_loc = x.shape
    _, n_col = w_mat.shape
    m_blk = m_glob // N_DEV
    n_half = n_col // 2

    n_q = n_half // 2

    def body(x_ref, w_ref, out_ref, acc_ref, send_sems, recv_sems,
             credit_sems, ax_send, ax_recv, ax_send_sems, ax_recv_sems):
        d = lax.axis_index("i")
        rk = _rank_of_pos(d)
        fwd = _pos_of_rank(rk + 1)
        bwd = _pos_of_rank(rk + N_DEV - 1)

        barrier_sem = pltpu.get_barrier_semaphore()
        for nbr in (bwd, fwd):
            _sem_signal(barrier_sem, inc=1, device_id=(nbr,),
                        device_id_type=pl.DeviceIdType.MESH)
        _sem_wait(barrier_sem, 2)

        def chunk_half(j, col0):
            xc = x_ref[pl.ds(j * m_blk, m_blk), :]
            return lax.dot_general(
                xc, w_ref[:, col0:col0 + n_half], (((1,), (0,)), ((), ())),
                precision=lax.Precision.HIGHEST,
                preferred_element_type=jnp.float32,
            )

        rings = (
            dict(col0=0, to=fwd, writer=bwd, jsign=-1),
            dict(col0=n_half, to=bwd, writer=fwd, jsign=+1),
        )
        stripes = []
        for q in range(2):
            for r, rg in enumerate(rings):
                stripes.append(dict(
                    sid=2 * q + r, r=r, q=q, rg=rg,
                    col0=rg["col0"] + q * n_q,
                ))

        def mk_rdma(st, s):
            return pltpu.make_async_remote_copy(
                src_ref=acc_ref.at[st["sid"], (s - 1) % 2],
                dst_ref=acc_ref.at[st["sid"], s % 2],
                send_sem=send_sems.at[st["sid"], (s - 1) % 2],
                recv_sem=recv_sems.at[st["sid"], s % 2],
                device_id=(st["rg"]["to"],),
                device_id_type=pl.DeviceIdType.MESH,
            )

        cseed = [
            chunk_half(_pos_of_rank(rk + rg["jsign"] + N_DEV), rg["col0"])
            for rg in rings
        ]
        for st in stripes:
            acc_ref[st["sid"], 0, :, :] = (
                cseed[st["r"]][:, st["q"] * n_q:(st["q"] + 1) * n_q])

        rdmas = {}
        for st in stripes:
            rdmas[st["sid"]] = mk_rdma(st, 1)
            rdmas[st["sid"]].start()

        amax_parts = []
        for s in range(1, N_DEV):
            src = (s - 1) % 2
            dst = s % 2
            cs = [
                chunk_half(
                    _pos_of_rank(rk + rg["jsign"] * (s + 1) + 2 * N_DEV),
                    rg["col0"])
                for rg in rings
            ]
            for st in stripes:
                rdma = rdmas[st["sid"]]
                rdma.wait_recv()
                rdma.wait_send()
                if s <= N_DEV - 2:
                    _sem_signal(credit_sems.at[st["sid"], src], inc=1,
                                device_id=(st["rg"]["writer"],),
                                device_id_type=pl.DeviceIdType.MESH)
                c = cs[st["r"]][:, st["q"] * n_q:(st["q"] + 1) * n_q]
                if s < N_DEV - 1:
                    acc_ref[st["sid"], dst, :, :] = (
                        acc_ref[st["sid"], dst, :, :] + c)
                    _sem_wait(credit_sems.at[st["sid"], src], 1)
                    nxt = mk_rdma(st, s + 1)
                    nxt.start()
                    rdmas[st["sid"]] = nxt
                else:
                    y_st = acc_ref[st["sid"], dst, :, :] + c
                    out_ref[:, st["col0"]:st["col0"] + n_q] = y_st
                    amax_parts.append(jnp.max(jnp.abs(y_st)))

        amax = jnp.maximum(jnp.maximum(amax_parts[0], amax_parts[1]),
                           jnp.maximum(amax_parts[2], amax_parts[3]))

        ax_send[:, :] = jnp.full((8, 128), amax, jnp.float32)
        ax_recv[0, :, :] = jnp.full((8, 128), amax, jnp.float32)
        ax_rdmas = []
        for o in range(1, N_DEV):
            tgt = lax.rem(d + o, N_DEV)
            ax = pltpu.make_async_remote_copy(
                src_ref=ax_send,
                dst_ref=ax_recv.at[o],
                send_sem=ax_send_sems.at[o],
                recv_sem=ax_recv_sems.at[o],
                device_id=(tgt,),
                device_id_type=pl.DeviceIdType.MESH,
            )
            ax.start()
            ax_rdmas.append(ax)
        for ax in ax_rdmas:
            ax.wait_recv()
        amax = jnp.max(ax_recv[:, :, :])
        for ax in ax_rdmas:
            ax.wait_send()

        scale = amax / 127.0
        q = jnp.clip(jnp.round(out_ref[:, :] / scale), -127.0, 127.0)
        out_ref[:, :] = q * scale

    return pl.pallas_call(
        body,
        out_shape=jax.ShapeDtypeStruct((m_blk, n_col), jnp.float32),
        in_specs=[
            pl.BlockSpec(memory_space=pltpu.VMEM),
            pl.BlockSpec(memory_space=pltpu.VMEM),
        ],
        out_specs=pl.BlockSpec(memory_space=pltpu.VMEM),
        scratch_shapes=[
            pltpu.VMEM((4, 2, m_blk, n_half // 2), jnp.float32),
            pltpu.SemaphoreType.DMA((4, 2)),
            pltpu.SemaphoreType.DMA((4, 2)),
            pltpu.SemaphoreType.REGULAR((4, 2)),
            pltpu.VMEM((8, 128), jnp.float32),
            pltpu.VMEM((N_DEV, 8, 128), jnp.float32),
            pltpu.SemaphoreType.DMA((N_DEV,)),
            pltpu.SemaphoreType.DMA((N_DEV,)),
        ],
        compiler_params=pltpu.CompilerParams(collective_id=0),
    )(x, w_mat)


# device time: 719196 ns/iter; 2.2724x vs baseline; 1.0026x over previous
import jax
import jax.numpy as jnp
from jax import lax
from jax.experimental import pallas as pl
from jax.experimental.pallas import tpu as pltpu

N_DEV = 32
LOG2_N = 5

_sem_signal = getattr(pl, "semaphore_signal", None) or pltpu.semaphore_signal
_sem_wait = getattr(pl, "semaphore_wait", None) or pltpu.semaphore_wait


def _coords_of_pos(p):
    z = p // 8
    idx = p % 8
    y = idx // 2
    xb = idx % 2
    x = jnp.where(y % 2 == 0, xb, 1 - xb)
    return x, y, z


def _rank_of_pos(p):
    x, y, z = _coords_of_pos(p)
    pr = 4 * z + jnp.where(z % 2 == 0, y, 3 - y)
    return jnp.where(x == 0, pr, 31 - pr)


def _pos_of_rank(rr):
    rr = jnp.remainder(rr, N_DEV)
    x = jnp.where(rr < 16, 0, 1)
    pr = jnp.where(rr < 16, rr, 31 - rr)
    z = pr // 4
    yy = pr % 4
    y = jnp.where(z % 2 == 0, yy, 3 - yy)
    idx = 2 * y + jnp.where(y % 2 == 0, x, 1 - x)
    return 8 * z + idx


def kernel(x, w_mat):
    m_glob, k_loc = x.shape
    _, n_col = w_mat.shape
    m_blk = m_glob // N_DEV
    n_half = n_col // 2

    n_q = n_half // 2

    def body(x_ref, w_ref, out_ref, acc_ref, send_sems, recv_sems,
             credit_sems, ax_send, ax_recv, ax_send_sems, ax_recv_sems):
        d = lax.axis_index("i")
        rk = _rank_of_pos(d)
        fwd = _pos_of_rank(rk + 1)
        bwd = _pos_of_rank(rk + N_DEV - 1)

        barrier_sem = pltpu.get_barrier_semaphore()
        for nbr in (bwd, fwd):
            _sem_signal(barrier_sem, inc=1, device_id=(nbr,),
                        device_id_type=pl.DeviceIdType.MESH)
        _sem_wait(barrier_sem, 2)

        def chunk_w(j, col0, width):
            xc = x_ref[pl.ds(j * m_blk, m_blk), :]
            return lax.dot_general(
                xc, w_ref[:, col0:col0 + width], (((1,), (0,)), ((), ())),
                precision=lax.Precision.HIGHEST,
                preferred_element_type=jnp.float32,
            )

        def chunk_half(j, col0):
            return chunk_w(j, col0, n_half)

        rings = (
            dict(col0=0, to=fwd, writer=bwd, jsign=-1),
            dict(col0=n_half, to=bwd, writer=fwd, jsign=+1),
        )
        stripes = []
        for q in range(2):
            for r, rg in enumerate(rings):
                stripes.append(dict(
                    sid=2 * q + r, r=r, q=q, rg=rg,
                    col0=rg["col0"] + q * n_q,
                ))

        def mk_rdma(st, s):
            return pltpu.make_async_remote_copy(
                src_ref=acc_ref.at[st["sid"], (s - 1) % 2],
                dst_ref=acc_ref.at[st["sid"], s % 2],
                send_sem=send_sems.at[st["sid"], (s - 1) % 2],
                recv_sem=recv_sems.at[st["sid"], s % 2],
                device_id=(st["rg"]["to"],),
                device_id_type=pl.DeviceIdType.MESH,
            )

        rdmas = {}
        for st in stripes:
            j0 = _pos_of_rank(rk + st["rg"]["jsign"] + N_DEV)
            acc_ref[st["sid"], 0, :, :] = chunk_w(j0, st["col0"], n_q)
            rdmas[st["sid"]] = mk_rdma(st, 1)
            rdmas[st["sid"]].start()

        amax_parts = []
        for s in range(1, N_DEV):
            src = (s - 1) % 2
            dst = s % 2
            cs = [
                chunk_half(
                    _pos_of_rank(rk + rg["jsign"] * (s + 1) + 2 * N_DEV),
                    rg["col0"])
                for rg in rings
            ]
            for st in stripes:
                rdma = rdmas[st["sid"]]
                rdma.wait_recv()
                rdma.wait_send()
                if s <= N_DEV - 2:
                    _sem_signal(credit_sems.at[st["sid"], src], inc=1,
                                device_id=(st["rg"]["writer"],),
                                device_id_type=pl.DeviceIdType.MESH)
                c = cs[st["r"]][:, st["q"] * n_q:(st["q"] + 1) * n_q]
                if s < N_DEV - 1:
                    acc_ref[st["sid"], dst, :, :] = (
                        acc_ref[st["sid"], dst, :, :] + c)
                    _sem_wait(credit_sems.at[st["sid"], src], 1)
                    nxt = mk_rdma(st, s + 1)
                    nxt.start()
                    rdmas[st["sid"]] = nxt
                else:
                    y_st = acc_ref[st["sid"], dst, :, :] + c
                    out_ref[:, st["col0"]:st["col0"] + n_q] = y_st
                    amax_parts.append(jnp.max(jnp.abs(y_st)))

        amax = jnp.maximum(jnp.maximum(amax_parts[0], amax_parts[1]),
                           jnp.maximum(amax_parts[2], amax_parts[3]))

        ax_send[:, :] = jnp.full((8, 128), amax, jnp.float32)
        ax_recv[0, :, :] = jnp.full((8, 128), amax, jnp.float32)
        ax_rdmas = []
        for o in range(1, N_DEV):
            tgt = lax.rem(d + o, N_DEV)
            ax = pltpu.make_async_remote_copy(
                src_ref=ax_send,
                dst_ref=ax_recv.at[o],
                send_sem=ax_send_sems.at[o],
                recv_sem=ax_recv_sems.at[o],
                device_id=(tgt,),
                device_id_type=pl.DeviceIdType.MESH,
            )
            ax.start()
            ax_rdmas.append(ax)
        for ax in ax_rdmas:
            ax.wait_recv()
        amax = jnp.max(ax_recv[:, :, :])
        for ax in ax_rdmas:
            ax.wait_send()

        scale = amax / 127.0
        q = jnp.clip(jnp.round(out_ref[:, :] / scale), -127.0, 127.0)
        out_ref[:, :] = q * scale

    return pl.pallas_call(
        body,
        out_shape=jax.ShapeDtypeStruct((m_blk, n_col), jnp.float32),
        in_specs=[
            pl.BlockSpec(memory_space=pltpu.VMEM),
            pl.BlockSpec(memory_space=pltpu.VMEM),
        ],
        out_specs=pl.BlockSpec(memory_space=pltpu.VMEM),
        scratch_shapes=[
            pltpu.VMEM((4, 2, m_blk, n_half // 2), jnp.float32),
            pltpu.SemaphoreType.DMA((4, 2)),
            pltpu.SemaphoreType.DMA((4, 2)),
            pltpu.SemaphoreType.REGULAR((4, 2)),
            pltpu.VMEM((8, 128), jnp.float32),
            pltpu.VMEM((N_DEV, 8, 128), jnp.float32),
            pltpu.SemaphoreType.DMA((N_DEV,)),
            pltpu.SemaphoreType.DMA((N_DEV,)),
        ],
        compiler_params=pltpu.CompilerParams(collective_id=0),
    )(x, w_mat)
